# Initial kernel scaffold; baseline (speedup 1.0000x reference)
import jax
import jax.numpy as jnp
from jax.experimental import pallas as pl


def kernel(x, edge_index, W1, b1, W2, b2):
    raise NotImplementedError("write your pallas kernel here")


# speedup vs baseline: 44.0557x; 44.0557x over previous
import functools

import jax
import jax.numpy as jnp
from jax import lax
from jax.experimental import pallas as pl
from jax.experimental.pallas import tpu as pltpu
from jax.experimental.pallas import tpu_sc as plsc

N = 10000
NP = 10240
E = 320000
EP = 327680
B = 128
STEPS = EP // (32 * B)
GS = 8
NC, NS = 2, 16
RPT = NP // NS
IN_D = 128
HID = 16
OUT_D = 7

def _deg_body(dst_hbm, ones_hbm, zeros_hbm, out_hbm, idx_v, ones_v, acc):
    cid = lax.axis_index("c")
    sid = lax.axis_index("s")
    wid = sid * NC + cid
    pltpu.sync_copy(zeros_hbm.at[pl.ds(sid * RPT, RPT)],
                    acc.at[pl.ds(sid * RPT, RPT)])
    pltpu.sync_copy(ones_hbm, ones_v)
    pltpu.sync_copy(dst_hbm.at[pl.ds(wid * STEPS, STEPS)], idx_v)
    plsc.subcore_barrier()

    def step(j, carry):
        pltpu.sync_copy(ones_v, acc.at[idx_v.at[j]], add=True)
        return carry

    lax.fori_loop(0, STEPS, step, 0)
    plsc.subcore_barrier()
    pltpu.sync_copy(acc.at[pl.ds(sid * RPT, RPT)],
                    out_hbm.at[cid, pl.ds(sid * RPT, RPT)])


def _agg_body(table_hbm, src_hbm, dst_hbm, zeros_hbm, out_hbm,
              src_v, dst_v, rows_v, acc, sem):
    cid = lax.axis_index("c")
    sid = lax.axis_index("s")
    wid = sid * NC + cid
    pltpu.sync_copy(zeros_hbm.at[pl.ds(sid * RPT, RPT)],
                    acc.at[pl.ds(sid * RPT, RPT)])
    pltpu.sync_copy(src_hbm.at[pl.ds(wid * STEPS, STEPS)], src_v)
    pltpu.sync_copy(dst_hbm.at[pl.ds(wid * STEPS, STEPS)], dst_v)
    plsc.subcore_barrier()

    def group(g, carry):
        base = g * GS
        descs = [
            pltpu.async_copy(table_hbm.at[src_v.at[base + i]], rows_v.at[i], sem)
            for i in range(GS)
        ]
        for d in descs:
            d.wait()
        for i in range(GS):
            pltpu.sync_copy(rows_v.at[i], acc.at[dst_v.at[base + i]], add=True)
        return carry

    lax.fori_loop(0, STEPS // GS, group, 0)
    plsc.subcore_barrier()
    pltpu.sync_copy(acc.at[pl.ds(sid * RPT, RPT)],
                    out_hbm.at[cid, pl.ds(sid * RPT, RPT)])


@functools.cache
def _sc_calls():
    mesh = plsc.VectorSubcoreMesh(core_axis_name="c", subcore_axis_name="s",
                                  num_cores=NC, num_subcores=NS)
    params = pltpu.CompilerParams(use_tc_tiling_on_sc=False)
    deg_call = pl.kernel(
        _deg_body,
        out_type=jax.ShapeDtypeStruct((NC, NP, HID), jnp.float32),
        mesh=mesh,
        compiler_params=params,
        scratch_types=[
            pltpu.VMEM((STEPS, B), jnp.int32),
            pltpu.VMEM((B, HID), jnp.float32),
            pltpu.VMEM_SHARED((NP, HID), jnp.float32),
        ],
    )
    agg_call = pl.kernel(
        _agg_body,
        out_type=jax.ShapeDtypeStruct((NC, NP, HID), jnp.float32),
        mesh=mesh,
        compiler_params=params,
        scratch_types=[
            pltpu.VMEM((STEPS, B), jnp.int32),
            pltpu.VMEM((STEPS, B), jnp.int32),
            pltpu.VMEM((GS, B, HID), jnp.float32),
            pltpu.VMEM_SHARED((NP, HID), jnp.float32),
            pltpu.SemaphoreType.DMA,
        ],
    )
    return deg_call, agg_call


def _tc1_body(x_ref, w1_ref, d0_ref, d1_ref, xs_ref, dis_ref):
    deg = d0_ref[0:NP, 0:1] + d1_ref[0:NP, 0:1] + 1.0
    dis = lax.rsqrt(deg)
    xw = jnp.dot(x_ref[...], w1_ref[...], preferred_element_type=jnp.float32)
    xs_ref[...] = dis * xw
    dis_ref[...] = dis


_tc1 = pl.pallas_call(
    _tc1_body,
    out_shape=(jax.ShapeDtypeStruct((NP, HID), jnp.float32),
               jax.ShapeDtypeStruct((NP, 1), jnp.float32)),
)


def _tc2_body(a0_ref, a1_ref, xs1_ref, dis_ref, w2_ref, b1_ref, xs2_ref):
    dis = dis_ref[...]
    h = dis * (a0_ref[...] + a1_ref[...] + xs1_ref[...]) + b1_ref[...]
    h = jnp.maximum(h, 0.0)
    xs2_ref[...] = dis * jnp.dot(h, w2_ref[...],
                                 preferred_element_type=jnp.float32)


_tc2 = pl.pallas_call(
    _tc2_body,
    out_shape=jax.ShapeDtypeStruct((NP, HID), jnp.float32),
)


def _tc3_body(a0_ref, a1_ref, xs2_ref, dis_ref, b2_ref, out_ref):
    dis = dis_ref[0:N]
    o = dis * (a0_ref[0:N, 0:OUT_D] + a1_ref[0:N, 0:OUT_D]
               + xs2_ref[0:N, 0:OUT_D]) + b2_ref[...]
    m = jnp.max(o, axis=1, keepdims=True)
    z = o - m
    out_ref[...] = z - jnp.log(jnp.sum(jnp.exp(z), axis=1, keepdims=True))


_tc3 = pl.pallas_call(
    _tc3_body,
    out_shape=jax.ShapeDtypeStruct((N, OUT_D), jnp.float32),
)


def kernel(x, edge_index, W1, b1, W2, b2):
    src = edge_index[0]
    dst = edge_index[1]
    pad = (N + (jnp.arange(EP - E, dtype=jnp.int32) % (NP - N))).astype(jnp.int32)
    src_p = jnp.concatenate([src, pad]).reshape(EP // B, B)
    dst_p = jnp.concatenate([dst, pad]).reshape(EP // B, B)
    x_p = jnp.concatenate(
        [x, jnp.zeros((NP - N, IN_D), jnp.float32)], axis=0)
    w2p = jnp.zeros((HID, HID), jnp.float32).at[:, :OUT_D].set(W2)
    zeros16 = jnp.zeros((NP, HID), jnp.float32)
    ones_b = jnp.zeros((B, HID), jnp.float32).at[:, 0].set(1.0)

    _deg_call, _agg_call = _sc_calls()
    degs = _deg_call(dst_p, ones_b, zeros16)
    xs1, dis = _tc1(x_p, W1, degs[0], degs[1])
    agg1 = _agg_call(xs1, src_p, dst_p, zeros16)
    xs2 = _tc2(agg1[0], agg1[1], xs1, dis, w2p,
               b1.reshape(1, HID))
    agg2 = _agg_call(xs2, src_p, dst_p, zeros16)
    return _tc3(agg2[0], agg2[1], xs2, dis, b2.reshape(1, OUT_D))


# speedup vs baseline: 56.4939x; 1.2823x over previous
import functools

import jax
import jax.numpy as jnp
from jax import lax
from jax.experimental import pallas as pl
from jax.experimental.pallas import tpu as pltpu
from jax.experimental.pallas import tpu_sc as plsc

N = 10000
NP = 10240
E = 320000
EP = 327680
B = 128
STEPS = EP // (32 * B)
GS = 8
NC, NS = 2, 16
RPT = NP // NS
IN_D = 128
HID = 16
OUT_D = 7

def _deg_body(dst_hbm, ones_hbm, zeros_hbm, out_hbm, idx_v, ones_v, acc, sem):
    cid = lax.axis_index("c")
    sid = lax.axis_index("s")
    wid = sid * NC + cid
    pltpu.sync_copy(zeros_hbm.at[pl.ds(sid * RPT, RPT)],
                    acc.at[pl.ds(sid * RPT, RPT)])
    pltpu.sync_copy(ones_hbm, ones_v)
    pltpu.sync_copy(dst_hbm.at[pl.ds(wid * STEPS, STEPS)], idx_v)
    plsc.subcore_barrier()

    def step(j, carry):
        pltpu.async_copy(ones_v, acc.at[idx_v.at[j]], sem, add=True)
        return carry

    lax.fori_loop(0, STEPS, step, 0)

    def drain(j, carry):
        pltpu.make_async_copy(ones_v, acc.at[idx_v.at[j]], sem).wait()
        return carry

    lax.fori_loop(0, STEPS, drain, 0)
    plsc.subcore_barrier()
    pltpu.sync_copy(acc.at[pl.ds(sid * RPT, RPT)],
                    out_hbm.at[cid, pl.ds(sid * RPT, RPT)])


def _agg_body(table_hbm, src_hbm, dst_hbm, zeros_hbm, out_hbm,
              src_v, dst_v, rows_v, acc, sem):
    cid = lax.axis_index("c")
    sid = lax.axis_index("s")
    wid = sid * NC + cid
    pltpu.sync_copy(zeros_hbm.at[pl.ds(sid * RPT, RPT)],
                    acc.at[pl.ds(sid * RPT, RPT)])
    pltpu.sync_copy(src_hbm.at[pl.ds(wid * STEPS, STEPS)], src_v)
    pltpu.sync_copy(dst_hbm.at[pl.ds(wid * STEPS, STEPS)], dst_v)
    plsc.subcore_barrier()

    ngrp = STEPS // GS

    def fire(g, b):
        for i in range(GS):
            pltpu.async_copy(table_hbm.at[src_v.at[g * GS + i]],
                             rows_v.at[b, i], sem)

    def wait_fired(g, b):
        for i in range(GS):
            pltpu.make_async_copy(table_hbm.at[src_v.at[g * GS + i]],
                                  rows_v.at[b, i], sem).wait()

    fire(0, 0)

    def group(g, carry):
        b = g % 2

        @pl.when(g + 1 < ngrp)
        def _():
            fire(g + 1, 1 - b)

        wait_fired(g, b)
        for i in range(GS):
            pltpu.sync_copy(rows_v.at[b, i], acc.at[dst_v.at[g * GS + i]],
                            add=True)
        return carry

    lax.fori_loop(0, ngrp, group, 0)
    plsc.subcore_barrier()
    pltpu.sync_copy(acc.at[pl.ds(sid * RPT, RPT)],
                    out_hbm.at[cid, pl.ds(sid * RPT, RPT)])


@functools.cache
def _sc_calls():
    mesh = plsc.VectorSubcoreMesh(core_axis_name="c", subcore_axis_name="s",
                                  num_cores=NC, num_subcores=NS)
    params = pltpu.CompilerParams(use_tc_tiling_on_sc=False)
    deg_call = pl.kernel(
        _deg_body,
        out_type=jax.ShapeDtypeStruct((NC, NP, HID), jnp.float32),
        mesh=mesh,
        compiler_params=params,
        scratch_types=[
            pltpu.VMEM((STEPS, B), jnp.int32),
            pltpu.VMEM((B, HID), jnp.float32),
            pltpu.VMEM_SHARED((NP, HID), jnp.float32),
            pltpu.SemaphoreType.DMA,
        ],
    )
    agg_call = pl.kernel(
        _agg_body,
        out_type=jax.ShapeDtypeStruct((NC, NP, HID), jnp.float32),
        mesh=mesh,
        compiler_params=params,
        scratch_types=[
            pltpu.VMEM((STEPS, B), jnp.int32),
            pltpu.VMEM((STEPS, B), jnp.int32),
            pltpu.VMEM((2, GS, B, HID), jnp.float32),
            pltpu.VMEM_SHARED((NP, HID), jnp.float32),
            pltpu.SemaphoreType.DMA,
        ],
    )
    return deg_call, agg_call


def _tc1_body(x_ref, w1_ref, d_ref, xs_ref, dis_ref):
    deg = d_ref[0, 0:NP, 0:1] + d_ref[1, 0:NP, 0:1] + 1.0
    dis = lax.rsqrt(deg)
    xw = jnp.dot(x_ref[...], w1_ref[...], preferred_element_type=jnp.float32)
    xs_ref[...] = dis * xw
    dis_ref[...] = dis


_tc1 = pl.pallas_call(
    _tc1_body,
    out_shape=(jax.ShapeDtypeStruct((NP, HID), jnp.float32),
               jax.ShapeDtypeStruct((NP, 1), jnp.float32)),
)


def _tc2_body(a_ref, xs1_ref, dis_ref, w2_ref, b1_ref, xs2_ref):
    dis = dis_ref[...]
    h = dis * (a_ref[0] + a_ref[1] + xs1_ref[...]) + b1_ref[...]
    h = jnp.maximum(h, 0.0)
    xs2_ref[...] = dis * jnp.dot(h, w2_ref[...],
                                 preferred_element_type=jnp.float32)


_tc2 = pl.pallas_call(
    _tc2_body,
    out_shape=jax.ShapeDtypeStruct((NP, HID), jnp.float32),
)


def _tc3_body(a_ref, xs2_ref, dis_ref, b2_ref, out_ref):
    dis = dis_ref[0:N]
    o = dis * (a_ref[0, 0:N, 0:OUT_D] + a_ref[1, 0:N, 0:OUT_D]
               + xs2_ref[0:N, 0:OUT_D]) + b2_ref[...]
    m = jnp.max(o, axis=1, keepdims=True)
    z = o - m
    out_ref[...] = z - jnp.log(jnp.sum(jnp.exp(z), axis=1, keepdims=True))


_tc3 = pl.pallas_call(
    _tc3_body,
    out_shape=jax.ShapeDtypeStruct((N, OUT_D), jnp.float32),
)


def kernel(x, edge_index, W1, b1, W2, b2):
    src = edge_index[0]
    dst = edge_index[1]
    pad = (N + (jnp.arange(EP - E, dtype=jnp.int32) % (NP - N))).astype(jnp.int32)
    src_p = jnp.concatenate([src, pad]).reshape(EP // B, B)
    dst_p = jnp.concatenate([dst, pad]).reshape(EP // B, B)
    x_p = jnp.concatenate(
        [x, jnp.zeros((NP - N, IN_D), jnp.float32)], axis=0)
    w2p = jnp.zeros((HID, HID), jnp.float32).at[:, :OUT_D].set(W2)
    zeros16 = jnp.zeros((NP, HID), jnp.float32)
    ones_b = jnp.ones((B, HID), jnp.float32)

    _deg_call, _agg_call = _sc_calls()
    degs = _deg_call(dst_p, ones_b, zeros16)
    xs1, dis = _tc1(x_p, W1, degs)
    agg1 = _agg_call(xs1, src_p, dst_p, zeros16)
    xs2 = _tc2(agg1, xs1, dis, w2p, b1.reshape(1, HID))
    agg2 = _agg_call(xs2, src_p, dst_p, zeros16)
    return _tc3(agg2, xs2, dis, b2.reshape(1, OUT_D))


# speedup vs baseline: 84.0169x; 1.4872x over previous
import functools

import jax
import jax.numpy as jnp
from jax import lax
from jax.experimental import pallas as pl
from jax.experimental.pallas import tpu as pltpu
from jax.experimental.pallas import tpu_sc as plsc

N = 10000
NP = 10240
NB = NP // 8
E = 320000
B = 128
EP = 327680
STEPS = EP // (32 * B)
GS = 8
NC, NS = 2, 16
RPT = NP // NS
IN_D = 128
HID = 16
OUT_D = 7


def _deg_body(ei_hbm, ones_hbm, zeros_hbm, out_hbm, ei_v, ones_v, acc, sem):
    cid = lax.axis_index("c")
    sid = lax.axis_index("s")
    wid = sid * NC + cid
    pltpu.sync_copy(zeros_hbm.at[pl.ds(sid * RPT, RPT)],
                    acc.at[pl.ds(sid * RPT, RPT)])
    pltpu.sync_copy(ones_hbm, ones_v)
    pltpu.sync_copy(ei_hbm.at[pl.ds(wid * STEPS, STEPS)], ei_v)
    plsc.subcore_barrier()

    def step(j, carry):
        pltpu.async_copy(ones_v, acc.at[ei_v.at[j, 1]], sem, add=True)
        return carry

    lax.fori_loop(0, STEPS, step, 0)

    def drain(j, carry):
        pltpu.make_async_copy(ones_v, acc.at[ei_v.at[j, 1]], sem).wait()
        return carry

    lax.fori_loop(0, STEPS, drain, 0)
    plsc.subcore_barrier()
    pltpu.sync_copy(acc.at[pl.ds(sid * RPT, RPT)],
                    out_hbm.at[cid, pl.ds(sid * RPT, RPT)])


def _agg_body(table_hbm, ei_hbm, zeros_hbm, out_hbm, ei_v, rows_v, acc, sem):
    cid = lax.axis_index("c")
    sid = lax.axis_index("s")
    wid = sid * NC + cid
    pltpu.sync_copy(zeros_hbm.at[pl.ds(sid * RPT, RPT)],
                    acc.at[pl.ds(sid * RPT, RPT)])
    pltpu.sync_copy(ei_hbm.at[pl.ds(wid * STEPS, STEPS)], ei_v)
    plsc.subcore_barrier()

    ngrp = STEPS // GS

    def fire(g, b):
        for i in range(GS):
            pltpu.async_copy(table_hbm.at[ei_v.at[g * GS + i, 0]],
                             rows_v.at[b, i], sem)

    def wait_fired(g, b):
        for i in range(GS):
            pltpu.make_async_copy(table_hbm.at[ei_v.at[g * GS + i, 0]],
                                  rows_v.at[b, i], sem).wait()

    fire(0, 0)

    def group(g, carry):
        b = g % 2

        @pl.when(g + 1 < ngrp)
        def _():
            fire(g + 1, 1 - b)

        wait_fired(g, b)
        for i in range(GS):
            pltpu.sync_copy(rows_v.at[b, i], acc.at[ei_v.at[g * GS + i, 1]],
                            add=True)
        return carry

    lax.fori_loop(0, ngrp, group, 0)
    plsc.subcore_barrier()
    pltpu.sync_copy(acc.at[pl.ds(sid * RPT, RPT)],
                    out_hbm.at[cid, pl.ds(sid * RPT, RPT)])


@functools.cache
def _sc_calls():
    mesh = plsc.VectorSubcoreMesh(core_axis_name="c", subcore_axis_name="s",
                                  num_cores=NC, num_subcores=NS)
    params = pltpu.CompilerParams(use_tc_tiling_on_sc=False)
    deg_call = pl.kernel(
        _deg_body,
        out_type=jax.ShapeDtypeStruct((NC, NP, HID), jnp.float32),
        mesh=mesh,
        compiler_params=params,
        scratch_types=[
            pltpu.VMEM((STEPS, 2, B), jnp.int32),
            pltpu.VMEM((B, HID), jnp.float32),
            pltpu.VMEM_SHARED((NP, HID), jnp.float32),
            pltpu.SemaphoreType.DMA,
        ],
    )
    agg_call = pl.kernel(
        _agg_body,
        out_type=jax.ShapeDtypeStruct((NC, NP, HID), jnp.float32),
        mesh=mesh,
        compiler_params=params,
        scratch_types=[
            pltpu.VMEM((STEPS, 2, B), jnp.int32),
            pltpu.VMEM((2, GS, B, HID), jnp.float32),
            pltpu.VMEM_SHARED((NP, HID), jnp.float32),
            pltpu.SemaphoreType.DMA,
        ],
    )
    return deg_call, agg_call


def _tc1_body(xb_ref, w1bd_ref, d_ref, xs_ref, dis_ref):
    dis = lax.rsqrt(d_ref[0] + d_ref[1] + 1.0)
    xw = jnp.dot(xb_ref[...], w1bd_ref[...],
                 preferred_element_type=jnp.float32)
    xs_ref[...] = dis * xw
    dis_ref[...] = dis


_tc1 = pl.pallas_call(
    _tc1_body,
    out_shape=(jax.ShapeDtypeStruct((NB, 128), jnp.float32),
               jax.ShapeDtypeStruct((NB, 128), jnp.float32)),
)


def _tc2_body(a_ref, xs1_ref, dis_ref, w2bd_ref, b1_ref, xs2_ref):
    dis = dis_ref[...]
    h = dis * (a_ref[0] + a_ref[1] + xs1_ref[...]) + b1_ref[...]
    h = jnp.maximum(h, 0.0)
    xs2_ref[...] = dis * jnp.dot(h, w2bd_ref[...],
                                 preferred_element_type=jnp.float32)


_tc2 = pl.pallas_call(
    _tc2_body,
    out_shape=jax.ShapeDtypeStruct((NB, 128), jnp.float32),
)


def _roll_lanes(m, s):
    return jnp.concatenate([m[:, s:], m[:, :s]], axis=1)


def _tc3_body(a_ref, xs2_ref, dis_ref, b2_ref, p_ref, sg_ref, out_ref):
    dis = dis_ref[...]
    o = dis * (a_ref[0] + a_ref[1] + xs2_ref[...]) + b2_ref[...]
    lane = lax.broadcasted_iota(jnp.int32, (NB, 128), 1) % HID
    valid = lane < OUT_D
    m = jnp.where(valid, o, -1e30)
    for s in (1, 2, 4, 8):
        m = jnp.maximum(m, _roll_lanes(m, s))
    mb = jnp.dot(m, p_ref[...], preferred_element_type=jnp.float32)
    z = o - mb
    e = jnp.where(valid, jnp.exp(z), 0.0)
    sb = jnp.dot(e, sg_ref[...], preferred_element_type=jnp.float32)
    out_ref[...] = z - jnp.log(sb)


_tc3 = pl.pallas_call(
    _tc3_body,
    out_shape=jax.ShapeDtypeStruct((NB, 128), jnp.float32),
)


def kernel(x, edge_index, W1, b1, W2, b2):
    f32 = jnp.float32
    ei = edge_index.reshape(2, E // B, B).transpose(1, 0, 2)
    npad = 32 * STEPS - E // B
    pad_b = (N + (jnp.arange(npad * 2 * B, dtype=jnp.int32) % (NP - N))
             ).reshape(npad, 2, B)
    ei_p = jnp.concatenate([ei, pad_b], axis=0)

    x_b = jnp.concatenate(
        [x, jnp.zeros((NP - N, IN_D), f32)], axis=0).reshape(NB, 8 * IN_D)

    w1bd = jnp.zeros((8 * IN_D, 128), f32)
    w2bd = jnp.zeros((128, 128), f32)
    for b in range(8):
        w1bd = w1bd.at[b * IN_D:(b + 1) * IN_D, b * HID:(b + 1) * HID].set(W1)
        w2bd = w2bd.at[b * HID:b * HID + HID, b * HID:b * HID + OUT_D].set(W2)
    b1_b = jnp.tile(b1, 8).reshape(1, 128)
    b2_b = jnp.tile(jnp.concatenate([b2, jnp.zeros((HID - OUT_D,), f32)]),
                    8).reshape(1, 128)

    lanes = jnp.arange(128, dtype=jnp.int32)
    base = (lanes // HID) * HID
    p_mat = (lanes[:, None] == base[None, :]).astype(f32)
    sg_mat = (base[:, None] == base[None, :]).astype(f32)

    zeros16 = jnp.zeros((NP, HID), f32)
    ones_b = jnp.ones((B, HID), f32)

    _deg_call, _agg_call = _sc_calls()
    degs = _deg_call(ei_p, ones_b, zeros16)
    xs1, dis = _tc1(x_b, w1bd, degs.reshape(2, NB, 128))
    agg1 = _agg_call(xs1.reshape(NP, HID), ei_p, zeros16)
    xs2 = _tc2(agg1.reshape(2, NB, 128), xs1, dis, w2bd, b1_b)
    agg2 = _agg_call(xs2.reshape(NP, HID), ei_p, zeros16)
    out_b = _tc3(agg2.reshape(2, NB, 128), xs2, dis, b2_b, p_mat, sg_mat)
    return out_b.reshape(NP, HID)[:N, :OUT_D]
